# Initial kernel scaffold; baseline (speedup 1.0000x reference)
import jax
import jax.numpy as jnp
from jax.experimental import pallas as pl


def kernel(X, edge_index, edge_weight, edge1, edge2, iw1, w_ih1, w_hh1, b_ih1, b_hh1, iw2, w_ih2, w_hh2, b_ih2, b_hh2, lin0_w, lin0_b, lin1_w, lin1_b, lin2_w, lin2_b, bn0_g, bn0_b, bn1_g, bn1_b):
    raise NotImplementedError("write your pallas kernel here")


# speedup vs baseline: 6.3038x; 6.3038x over previous
import functools

import jax
import jax.numpy as jnp
from jax import lax
from jax.experimental import pallas as pl
from jax.experimental.pallas import tpu as pltpu
from jax.experimental.pallas import tpu_sc as plsc

NC = 2
NS = 16
NW = NC * NS
EB = 128


def _cdiv(a, b):
    return (a + b - 1) // b


def _mesh():
    return plsc.VectorSubcoreMesh(
        core_axis_name="c", subcore_axis_name="s", num_cores=NC, num_subcores=NS
    )


def _sc_deg(col, w, n_pad):
    e_pad = col.shape[0]
    ept = e_pad // NW
    nblk = ept // EB
    nrow = n_pad // 128

    @functools.partial(
        pl.kernel,
        mesh=_mesh(),
        compiler_params=pltpu.CompilerParams(needs_layout_passes=False),
        out_type=jax.ShapeDtypeStruct((NW, nrow, 128), jnp.float32),
        scratch_types=[
            pltpu.VMEM((EB,), jnp.int32),
            pltpu.VMEM((EB,), jnp.float32),
            pltpu.VMEM((nrow, 128), jnp.float32),
        ],
    )
    def k(col_hbm, w_hbm, out_hbm, cidx_v, w_v, deg_v):
        c = lax.axis_index("c")
        s = lax.axis_index("s")
        wid = s * NC + c
        z16 = jnp.zeros((16,), jnp.float32)

        def zbody(i, carry):
            for t in range(128 // 16):
                deg_v[i, pl.ds(t * 16, 16)] = z16
            return carry

        lax.fori_loop(0, nrow, zbody, 0)
        lane0 = lax.iota(jnp.int32, 16) == 0

        def blk(b, carry):
            base = wid * ept + b * EB
            pltpu.sync_copy(col_hbm.at[pl.ds(base, EB)], cidx_v)
            pltpu.sync_copy(w_hbm.at[pl.ds(base, EB)], w_v)

            def edge(j, carry2):
                idx = jnp.full((16,), j, jnp.int32)
                cj = plsc.load_gather(cidx_v, [idx])
                wj = plsc.load_gather(w_v, [idx])
                plsc.addupdate_scatter(
                    deg_v, [cj >> 7, cj & 127], wj, mask=lane0)
                return carry2

            lax.fori_loop(0, EB, edge, 0)
            return carry

        lax.fori_loop(0, nblk, blk, 0)
        pltpu.sync_copy(deg_v, out_hbm.at[wid])

    return k(col, w)


def _sc_norm(row, col, w, dis_flat):
    e_pad = row.shape[0]
    n_pad = dis_flat.shape[0]
    ept = e_pad // NW
    nblk = ept // EB

    @functools.partial(
        pl.kernel,
        mesh=_mesh(),
        compiler_params=pltpu.CompilerParams(needs_layout_passes=False),
        out_type=jax.ShapeDtypeStruct((e_pad,), jnp.float32),
        scratch_types=[
            pltpu.VMEM((n_pad,), jnp.float32),
            pltpu.VMEM((EB,), jnp.int32),
            pltpu.VMEM((EB,), jnp.int32),
            pltpu.VMEM((EB,), jnp.float32),
            pltpu.VMEM((EB,), jnp.float32),
        ],
    )
    def k(row_hbm, col_hbm, w_hbm, dis_hbm, out_hbm, dis_v, r_v, c_v, w_v, o_v):
        c = lax.axis_index("c")
        s = lax.axis_index("s")
        wid = s * NC + c
        pltpu.sync_copy(dis_hbm, dis_v)

        def blk(b, carry):
            base = wid * ept + b * EB
            pltpu.sync_copy(row_hbm.at[pl.ds(base, EB)], r_v)
            pltpu.sync_copy(col_hbm.at[pl.ds(base, EB)], c_v)
            pltpu.sync_copy(w_hbm.at[pl.ds(base, EB)], w_v)
            for t in range(EB // 16):
                sl = pl.ds(t * 16, 16)
                d1 = plsc.load_gather(dis_v, [r_v[sl]])
                d2 = plsc.load_gather(dis_v, [c_v[sl]])
                o_v[sl] = d1 * w_v[sl] * d2
            pltpu.sync_copy(o_v, out_hbm.at[pl.ds(base, EB)])
            return carry

        lax.fori_loop(0, nblk, blk, 0)

    return k(row, col, w, dis_flat)


def _sc_agg(xw, row, col, norm):
    n_pad, d = xw.shape
    e_pad = row.shape[0]
    ept = e_pad // NW
    nblk = ept // EB
    stripe = n_pad // NS

    @functools.partial(
        pl.kernel,
        mesh=_mesh(),
        compiler_params=pltpu.CompilerParams(needs_layout_passes=False),
        out_type=jax.ShapeDtypeStruct((NC, n_pad, d), jnp.float32),
        scratch_types=[
            pltpu.VMEM((EB,), jnp.int32),
            pltpu.VMEM((EB,), jnp.int32),
            pltpu.VMEM((EB,), jnp.float32),
            pltpu.VMEM((EB, d), jnp.float32),
            pltpu.VMEM_SHARED((n_pad, d), jnp.float32),
            pltpu.SemaphoreType.DMA,
        ],
    )
    def k(xw_hbm, row_hbm, col_hbm, nrm_hbm, out_hbm,
          ridx_v, cidx_v, nrm_v, rows_v, acc_sh, sem):
        c = lax.axis_index("c")
        s = lax.axis_index("s")
        wid = s * NC + c
        z16 = jnp.zeros((16,), jnp.float32)

        def zrow(j, carry):
            for t in range(d // 16):
                rows_v[j, pl.ds(t * 16, 16)] = z16
            return carry

        lax.fori_loop(0, EB, zrow, 0)

        def zstripe(i, carry):
            pltpu.sync_copy(rows_v, acc_sh.at[pl.ds(s * stripe + i * EB, EB)])
            return carry

        lax.fori_loop(0, stripe // EB, zstripe, 0)
        plsc.subcore_barrier()

        def blk(b, carry):
            base = wid * ept + b * EB
            pltpu.sync_copy(row_hbm.at[pl.ds(base, EB)], ridx_v)
            pltpu.sync_copy(col_hbm.at[pl.ds(base, EB)], cidx_v)
            pltpu.sync_copy(nrm_hbm.at[pl.ds(base, EB)], nrm_v)
            pltpu.async_copy(xw_hbm.at[ridx_v], rows_v, sem).wait()

            def scale(j, carry2):
                nb = plsc.load_gather(nrm_v, [jnp.full((16,), j, jnp.int32)])
                for t in range(d // 16):
                    rows_v[j, pl.ds(t * 16, 16)] = rows_v[j, pl.ds(t * 16, 16)] * nb
                return carry2

            lax.fori_loop(0, EB, scale, 0)
            pltpu.sync_copy(rows_v, acc_sh.at[cidx_v], add=True)
            return carry

        lax.fori_loop(0, nblk, blk, 0)
        plsc.subcore_barrier()
        pltpu.sync_copy(
            acc_sh.at[pl.ds(s * stripe, stripe)],
            out_hbm.at[c, pl.ds(s * stripe, stripe)],
        )

    return k(xw, row, col, norm)


def _sc_edge(h2, e1, e2):
    n_pad, d = h2.shape
    nq_pad = e1.shape[0]
    qpt = nq_pad // NW
    nblk = qpt // EB

    @functools.partial(
        pl.kernel,
        mesh=_mesh(),
        compiler_params=pltpu.CompilerParams(needs_layout_passes=False),
        out_type=jax.ShapeDtypeStruct((nq_pad, d), jnp.float32),
        scratch_types=[
            pltpu.VMEM((EB,), jnp.int32),
            pltpu.VMEM((EB,), jnp.int32),
            pltpu.VMEM((EB, d), jnp.float32),
            pltpu.VMEM((EB, d), jnp.float32),
            pltpu.SemaphoreType.DMA,
        ],
    )
    def k(h_hbm, e1_hbm, e2_hbm, out_hbm, i1_v, i2_v, a_v, b_v, sem):
        c = lax.axis_index("c")
        s = lax.axis_index("s")
        wid = s * NC + c

        def blk(b, carry):
            base = wid * qpt + b * EB
            pltpu.sync_copy(e1_hbm.at[pl.ds(base, EB)], i1_v)
            pltpu.sync_copy(e2_hbm.at[pl.ds(base, EB)], i2_v)
            pltpu.async_copy(h_hbm.at[i1_v], a_v, sem).wait()
            pltpu.async_copy(h_hbm.at[i2_v], b_v, sem).wait()

            def mul(j, carry2):
                for t in range(d // 16):
                    sl = pl.ds(t * 16, 16)
                    a_v[j, sl] = a_v[j, sl] * b_v[j, sl]
                return carry2

            lax.fori_loop(0, EB, mul, 0)
            pltpu.sync_copy(a_v, out_hbm.at[pl.ds(base, EB)])
            return carry

        lax.fori_loop(0, nblk, blk, 0)

    return k(h2, e1, e2)


def _tc_gru(iw1, wih1, whh1, bih1, bhh1, iw2, wih2, whh2, bih2, bhh2):
    d = iw1.shape[0]

    def body(iw1_r, wih1_r, whh1_r, bih1_r, bhh1_r,
             iw2_r, wih2_r, whh2_r, bih2_r, bhh2_r, w1_o, w2_o):
        def gru(iw, wih, whh, bih, bhh):
            cdims = (((1,), (1,)), ((), ()))
            gi = lax.dot_general(iw, wih, cdims,
                                 preferred_element_type=jnp.float32) + bih
            gh = lax.dot_general(iw, whh, cdims,
                                 preferred_element_type=jnp.float32) + bhh
            r = jax.nn.sigmoid(gi[:, :d] + gh[:, :d])
            z = jax.nn.sigmoid(gi[:, d:2 * d] + gh[:, d:2 * d])
            n = jnp.tanh(gi[:, 2 * d:] + r * gh[:, 2 * d:])
            return (1.0 - z) * n + z * iw

        w1_o[...] = gru(iw1_r[...], wih1_r[...], whh1_r[...], bih1_r[...], bhh1_r[...])
        w2_o[...] = gru(iw2_r[...], wih2_r[...], whh2_r[...], bih2_r[...], bhh2_r[...])

    return pl.pallas_call(
        body,
        out_shape=(jax.ShapeDtypeStruct((d, d), jnp.float32),
                   jax.ShapeDtypeStruct((d, d), jnp.float32)),
    )(iw1, wih1, whh1, bih1.reshape(1, -1), bhh1.reshape(1, -1),
      iw2, wih2, whh2, bih2.reshape(1, -1), bhh2.reshape(1, -1))


def _tc_prep(degp, xp, w1):
    nw_, nrow, _ = degp.shape
    n_pad, d = xp.shape
    blk = 1024
    g = n_pad // blk
    rb = blk // 128

    def body(degp_r, x_r, w1_r, dis_o, xw_o):
        dg = jnp.sum(degp_r[...], axis=0) + 1.0
        dis_o[...] = jnp.where(dg > 0, lax.rsqrt(jnp.maximum(dg, 1e-12)), 0.0)
        xw_o[...] = jnp.dot(x_r[...], w1_r[...], preferred_element_type=jnp.float32)

    return pl.pallas_call(
        body,
        grid=(g,),
        in_specs=[
            pl.BlockSpec((nw_, rb, 128), lambda i: (0, i, 0)),
            pl.BlockSpec((blk, d), lambda i: (i, 0)),
            pl.BlockSpec((d, d), lambda i: (0, 0)),
        ],
        out_specs=[
            pl.BlockSpec((rb, 128), lambda i: (i, 0)),
            pl.BlockSpec((blk, d), lambda i: (i, 0)),
        ],
        out_shape=(jax.ShapeDtypeStruct((nrow, 128), jnp.float32),
                   jax.ShapeDtypeStruct((n_pad, d), jnp.float32)),
    )(degp, xp, w1)


def _tc_comb(p0, p1, xw, dis_col, w2=None):
    n_pad, d = xw.shape
    blk = 256
    g = n_pad // blk

    def body_mm(p0_r, p1_r, xw_r, dis_r, w2_r, o_r):
        dv = dis_r[...]
        h = jnp.maximum(p0_r[...] + p1_r[...] + dv * dv * xw_r[...], 0.0)
        o_r[...] = jnp.dot(h, w2_r[...], preferred_element_type=jnp.float32)

    def body_plain(p0_r, p1_r, xw_r, dis_r, o_r):
        dv = dis_r[...]
        o_r[...] = jnp.maximum(p0_r[...] + p1_r[...] + dv * dv * xw_r[...], 0.0)

    in_specs = [
        pl.BlockSpec((blk, d), lambda i: (i, 0)),
        pl.BlockSpec((blk, d), lambda i: (i, 0)),
        pl.BlockSpec((blk, d), lambda i: (i, 0)),
        pl.BlockSpec((blk, 1), lambda i: (i, 0)),
    ]
    args = [p0, p1, xw, dis_col]
    if w2 is not None:
        in_specs.append(pl.BlockSpec((d, d), lambda i: (0, 0)))
        args.append(w2)
        body = body_mm
    else:
        body = body_plain

    return pl.pallas_call(
        body,
        grid=(g,),
        in_specs=in_specs,
        out_specs=pl.BlockSpec((blk, d), lambda i: (i, 0)),
        out_shape=jax.ShapeDtypeStruct((n_pad, d), jnp.float32),
    )(*args)


def _tc_mlp(q, w0, b0, w1, b1, w2, b2, g0, be0, g1, be1):
    nq, d = q.shape
    h = w0.shape[0]
    c = w2.shape[0]
    blk = 512
    g = nq // blk
    cdims = (((1,), (1,)), ((), ()))

    def body(q_r, w0_r, b0_r, w1_r, b1_r, w2_r, b2_r,
             g0_r, be0_r, g1_r, be1_r, o_r):
        s = 1.0 / jnp.sqrt(jnp.float32(1.0 + 1e-5))
        x = lax.dot_general(q_r[...], w0_r[...], cdims,
                            preferred_element_type=jnp.float32) + b0_r[...]
        x = x * (g0_r[...] * s) + be0_r[...]
        x = jnp.maximum(x, 0.0)
        x = lax.dot_general(x, w1_r[...], cdims,
                            preferred_element_type=jnp.float32) + b1_r[...]
        x = x * (g1_r[...] * s) + be1_r[...]
        x = jnp.maximum(x, 0.0)
        x = lax.dot_general(x, w2_r[...], cdims,
                            preferred_element_type=jnp.float32) + b2_r[...]
        m = jnp.max(x, axis=-1, keepdims=True)
        ex = jnp.exp(x - m)
        o_r[...] = x - m - jnp.log(jnp.sum(ex, axis=-1, keepdims=True))

    full = lambda a: pl.BlockSpec(a.shape, lambda i: tuple(0 for _ in a.shape))
    args = [w0, b0.reshape(1, -1), w1, b1.reshape(1, -1), w2, b2.reshape(1, -1),
            g0.reshape(1, -1), be0.reshape(1, -1), g1.reshape(1, -1),
            be1.reshape(1, -1)]
    return pl.pallas_call(
        body,
        grid=(g,),
        in_specs=[pl.BlockSpec((blk, d), lambda i: (i, 0))] + [full(a) for a in args],
        out_specs=pl.BlockSpec((blk, c), lambda i: (i, 0)),
        out_shape=jax.ShapeDtypeStruct((nq, c), jnp.float32),
    )(q, *args)


def kernel(X, edge_index, edge_weight, edge1, edge2,
           iw1, w_ih1, w_hh1, b_ih1, b_hh1,
           iw2, w_ih2, w_hh2, b_ih2, b_hh2,
           lin0_w, lin0_b, lin1_w, lin1_b, lin2_w, lin2_b,
           bn0_g, bn0_b, bn1_g, bn1_b):
    n, d = X.shape
    e = edge_weight.shape[0]
    nq = edge1.shape[0]

    n_pad = _cdiv(n, NS * EB) * NS * EB
    e_pad = _cdiv(e, NW * EB) * NW * EB
    nq_pad = _cdiv(nq, NW * EB) * NW * EB

    row = jnp.pad(edge_index[0].astype(jnp.int32), (0, e_pad - e))
    col = jnp.pad(edge_index[1].astype(jnp.int32), (0, e_pad - e))
    w = jnp.pad(edge_weight.astype(jnp.float32), (0, e_pad - e))
    e1 = jnp.pad(edge1.astype(jnp.int32), (0, nq_pad - nq))
    e2 = jnp.pad(edge2.astype(jnp.int32), (0, nq_pad - nq))
    xp = jnp.pad(X, ((0, n_pad - n), (0, 0)))

    degp = _sc_deg(col, w, n_pad)
    w1, w2 = _tc_gru(iw1, w_ih1, w_hh1, b_ih1, b_hh1,
                     iw2, w_ih2, w_hh2, b_ih2, b_hh2)
    dis2d, xw1 = _tc_prep(degp, xp, w1)
    dis_flat = dis2d.reshape(-1)
    dis_col = dis_flat.reshape(-1, 1)

    norm = _sc_norm(row, col, w, dis_flat)
    p = _sc_agg(xw1, row, col, norm)
    xw2 = _tc_comb(p[0], p[1], xw1, dis_col, w2)
    p2 = _sc_agg(xw2, row, col, norm)
    h2 = _tc_comb(p2[0], p2[1], xw2, dis_col)

    q = _sc_edge(h2, e1, e2)
    out = _tc_mlp(q, lin0_w, lin0_b, lin1_w, lin1_b, lin2_w, lin2_b,
                  bn0_g, bn0_b, bn1_g, bn1_b)
    return out[:nq]


# speedup vs baseline: 6.4371x; 1.0211x over previous
import functools

import jax
import jax.numpy as jnp
from jax import lax
from jax.experimental import pallas as pl
from jax.experimental.pallas import tpu as pltpu
from jax.experimental.pallas import tpu_sc as plsc

NC = 2
NS = 16
NW = NC * NS
EB = 128


def _cdiv(a, b):
    return (a + b - 1) // b


def _mesh():
    return plsc.VectorSubcoreMesh(
        core_axis_name="c", subcore_axis_name="s", num_cores=NC, num_subcores=NS
    )


def _sc_deg(col, w, n_pad):
    e_pad = col.shape[0]
    ept = e_pad // NW
    nblk = ept // EB
    nrow = n_pad // 128

    @functools.partial(
        pl.kernel,
        mesh=_mesh(),
        compiler_params=pltpu.CompilerParams(needs_layout_passes=False),
        out_type=jax.ShapeDtypeStruct((NW, nrow, 128), jnp.float32),
        scratch_types=[
            pltpu.VMEM((EB,), jnp.int32),
            pltpu.VMEM((EB,), jnp.float32),
            pltpu.VMEM((nrow, 128), jnp.float32),
        ],
    )
    def k(col_hbm, w_hbm, out_hbm, cidx_v, w_v, deg_v):
        c = lax.axis_index("c")
        s = lax.axis_index("s")
        wid = s * NC + c
        z16 = jnp.zeros((16,), jnp.float32)

        def zbody(i, carry):
            for t in range(128 // 16):
                deg_v[i, pl.ds(t * 16, 16)] = z16
            return carry

        lax.fori_loop(0, nrow, zbody, 0)
        lane0 = lax.iota(jnp.int32, 16) == 0

        def blk(b, carry):
            base = wid * ept + b * EB
            pltpu.sync_copy(col_hbm.at[pl.ds(base, EB)], cidx_v)
            pltpu.sync_copy(w_hbm.at[pl.ds(base, EB)], w_v)

            def edge(j, carry2):
                idx = jnp.full((16,), j, jnp.int32)
                cj = plsc.load_gather(cidx_v, [idx])
                wj = plsc.load_gather(w_v, [idx])
                plsc.addupdate_scatter(
                    deg_v, [cj >> 7, cj & 127], wj, mask=lane0)
                return carry2

            lax.fori_loop(0, EB, edge, 0)
            return carry

        lax.fori_loop(0, nblk, blk, 0)
        pltpu.sync_copy(deg_v, out_hbm.at[wid])

    return k(col, w)


def _sc_norm(row, col, w, dis_flat):
    e_pad = row.shape[0]
    n_pad = dis_flat.shape[0]
    ept = e_pad // NW
    nblk = ept // EB

    @functools.partial(
        pl.kernel,
        mesh=_mesh(),
        compiler_params=pltpu.CompilerParams(needs_layout_passes=False),
        out_type=jax.ShapeDtypeStruct((e_pad,), jnp.float32),
        scratch_types=[
            pltpu.VMEM((n_pad,), jnp.float32),
            pltpu.VMEM((EB,), jnp.int32),
            pltpu.VMEM((EB,), jnp.int32),
            pltpu.VMEM((EB,), jnp.float32),
            pltpu.VMEM((EB,), jnp.float32),
        ],
    )
    def k(row_hbm, col_hbm, w_hbm, dis_hbm, out_hbm, dis_v, r_v, c_v, w_v, o_v):
        c = lax.axis_index("c")
        s = lax.axis_index("s")
        wid = s * NC + c
        pltpu.sync_copy(dis_hbm, dis_v)

        def blk(b, carry):
            base = wid * ept + b * EB
            pltpu.sync_copy(row_hbm.at[pl.ds(base, EB)], r_v)
            pltpu.sync_copy(col_hbm.at[pl.ds(base, EB)], c_v)
            pltpu.sync_copy(w_hbm.at[pl.ds(base, EB)], w_v)
            for t in range(EB // 16):
                sl = pl.ds(t * 16, 16)
                d1 = plsc.load_gather(dis_v, [r_v[sl]])
                d2 = plsc.load_gather(dis_v, [c_v[sl]])
                o_v[sl] = d1 * w_v[sl] * d2
            pltpu.sync_copy(o_v, out_hbm.at[pl.ds(base, EB)])
            return carry

        lax.fori_loop(0, nblk, blk, 0)

    return k(row, col, w, dis_flat)


def _sc_agg(xw, row2, col2, nrm2):
    n_pad, d = xw.shape
    nblk = row2.shape[0] // NW
    stripe = n_pad // NS

    @functools.partial(
        pl.kernel,
        mesh=_mesh(),
        compiler_params=pltpu.CompilerParams(needs_layout_passes=False),
        out_type=jax.ShapeDtypeStruct((NC, n_pad, d), jnp.float32),
        scratch_types=[
            pltpu.VMEM((2, EB), jnp.int32),
            pltpu.VMEM((2, EB), jnp.int32),
            pltpu.VMEM((2, EB), jnp.float32),
            pltpu.VMEM((EB, d), jnp.float32),
            pltpu.VMEM((EB, d), jnp.float32),
            pltpu.VMEM_SHARED((n_pad, d), jnp.float32),
            pltpu.SemaphoreType.DMA,
            pltpu.SemaphoreType.DMA,
            pltpu.SemaphoreType.DMA,
            pltpu.SemaphoreType.DMA,
        ],
    )
    def k(xw_hbm, row_hbm, col_hbm, nrm_hbm, out_hbm,
          ridx_v, cidx_v, nrm_v, rows0, rows1, acc_sh,
          sem0, sem1, isem0, isem1):
        c = lax.axis_index("c")
        s = lax.axis_index("s")
        wid = s * NC + c
        base = wid * nblk
        z16 = jnp.zeros((16,), jnp.float32)
        isems = (isem0, isem1)

        def load_idx(b, slot):
            isem = isems[slot]
            pltpu.async_copy(row_hbm.at[base + b], ridx_v.at[slot], isem)
            pltpu.async_copy(col_hbm.at[base + b], cidx_v.at[slot], isem)
            pltpu.async_copy(nrm_hbm.at[base + b], nrm_v.at[slot], isem)

        def wait_idx(b, slot):
            isem = isems[slot]
            pltpu.make_async_copy(row_hbm.at[base + b], ridx_v.at[slot], isem).wait()
            pltpu.make_async_copy(col_hbm.at[base + b], cidx_v.at[slot], isem).wait()
            pltpu.make_async_copy(nrm_hbm.at[base + b], nrm_v.at[slot], isem).wait()

        load_idx(0, 0)
        load_idx(1, 1)

        def zrow(j, carry):
            for t in range(d // 16):
                rows0[j, pl.ds(t * 16, 16)] = z16
            return carry

        lax.fori_loop(0, EB, zrow, 0)

        def zstripe(i, carry):
            pltpu.sync_copy(rows0, acc_sh.at[pl.ds(s * stripe + i * EB, EB)])
            return carry

        lax.fori_loop(0, stripe // EB, zstripe, 0)
        wait_idx(0, 0)
        pltpu.async_copy(xw_hbm.at[ridx_v.at[0]], rows0, sem0)
        plsc.subcore_barrier()

        bufs = (rows0, rows1)
        sems = (sem0, sem1)

        def pair(bb, carry):
            for i in range(2):
                buf, other = bufs[i], bufs[1 - i]
                sem, osem = sems[i], sems[1 - i]
                b = bb * 2 + i

                @pl.when(b + 1 < nblk)
                def _():
                    wait_idx(b + 1, 1 - i)
                    pltpu.async_copy(xw_hbm.at[ridx_v.at[1 - i]], other, osem)

                pltpu.make_async_copy(xw_hbm.at[ridx_v.at[i]], buf, sem).wait()

                def scale(j4, carry2):
                    for u in range(4):
                        j = j4 * 4 + u
                        nb = plsc.load_gather(
                            nrm_v,
                            [jnp.full((16,), i, jnp.int32),
                             jnp.full((16,), j, jnp.int32)])
                        for t in range(d // 16):
                            buf[j, pl.ds(t * 16, 16)] = (
                                buf[j, pl.ds(t * 16, 16)] * nb)
                    return carry2

                lax.fori_loop(0, EB // 4, scale, 0)
                pltpu.sync_copy(buf, acc_sh.at[cidx_v.at[i]], add=True)

                @pl.when(b + 2 < nblk)
                def _():
                    load_idx(b + 2, i)
            return carry

        lax.fori_loop(0, nblk // 2, pair, 0)
        plsc.subcore_barrier()
        pltpu.sync_copy(
            acc_sh.at[pl.ds(s * stripe, stripe)],
            out_hbm.at[c, pl.ds(s * stripe, stripe)],
        )

    return k(xw, row2, col2, nrm2)


def _sc_edge(h2, e1, e2):
    n_pad, d = h2.shape
    nq_pad = e1.shape[0]
    qpt = nq_pad // NW
    nblk = qpt // EB

    @functools.partial(
        pl.kernel,
        mesh=_mesh(),
        compiler_params=pltpu.CompilerParams(needs_layout_passes=False),
        out_type=jax.ShapeDtypeStruct((nq_pad, d), jnp.float32),
        scratch_types=[
            pltpu.VMEM((EB,), jnp.int32),
            pltpu.VMEM((EB,), jnp.int32),
            pltpu.VMEM((EB, d), jnp.float32),
            pltpu.VMEM((EB, d), jnp.float32),
            pltpu.SemaphoreType.DMA,
        ],
    )
    def k(h_hbm, e1_hbm, e2_hbm, out_hbm, i1_v, i2_v, a_v, b_v, sem):
        c = lax.axis_index("c")
        s = lax.axis_index("s")
        wid = s * NC + c

        def blk(b, carry):
            base = wid * qpt + b * EB
            pltpu.sync_copy(e1_hbm.at[pl.ds(base, EB)], i1_v)
            pltpu.sync_copy(e2_hbm.at[pl.ds(base, EB)], i2_v)
            pltpu.async_copy(h_hbm.at[i1_v], a_v, sem).wait()
            pltpu.async_copy(h_hbm.at[i2_v], b_v, sem).wait()

            def mul(j, carry2):
                for t in range(d // 16):
                    sl = pl.ds(t * 16, 16)
                    a_v[j, sl] = a_v[j, sl] * b_v[j, sl]
                return carry2

            lax.fori_loop(0, EB, mul, 0)
            pltpu.sync_copy(a_v, out_hbm.at[pl.ds(base, EB)])
            return carry

        lax.fori_loop(0, nblk, blk, 0)

    return k(h2, e1, e2)


def _tc_gru(iw1, wih1, whh1, bih1, bhh1, iw2, wih2, whh2, bih2, bhh2):
    d = iw1.shape[0]

    def body(iw1_r, wih1_r, whh1_r, bih1_r, bhh1_r,
             iw2_r, wih2_r, whh2_r, bih2_r, bhh2_r, w1_o, w2_o):
        def gru(iw, wih, whh, bih, bhh):
            cdims = (((1,), (1,)), ((), ()))
            gi = lax.dot_general(iw, wih, cdims,
                                 preferred_element_type=jnp.float32) + bih
            gh = lax.dot_general(iw, whh, cdims,
                                 preferred_element_type=jnp.float32) + bhh
            r = jax.nn.sigmoid(gi[:, :d] + gh[:, :d])
            z = jax.nn.sigmoid(gi[:, d:2 * d] + gh[:, d:2 * d])
            n = jnp.tanh(gi[:, 2 * d:] + r * gh[:, 2 * d:])
            return (1.0 - z) * n + z * iw

        w1_o[...] = gru(iw1_r[...], wih1_r[...], whh1_r[...], bih1_r[...], bhh1_r[...])
        w2_o[...] = gru(iw2_r[...], wih2_r[...], whh2_r[...], bih2_r[...], bhh2_r[...])

    return pl.pallas_call(
        body,
        out_shape=(jax.ShapeDtypeStruct((d, d), jnp.float32),
                   jax.ShapeDtypeStruct((d, d), jnp.float32)),
    )(iw1, wih1, whh1, bih1.reshape(1, -1), bhh1.reshape(1, -1),
      iw2, wih2, whh2, bih2.reshape(1, -1), bhh2.reshape(1, -1))


def _tc_prep(degp, xp, w1):
    nw_, nrow, _ = degp.shape
    n_pad, d = xp.shape
    blk = 1024
    g = n_pad // blk
    rb = blk // 128

    def body(degp_r, x_r, w1_r, dis_o, xw_o):
        dg = jnp.sum(degp_r[...], axis=0) + 1.0
        dis_o[...] = jnp.where(dg > 0, lax.rsqrt(jnp.maximum(dg, 1e-12)), 0.0)
        xw_o[...] = jnp.dot(x_r[...], w1_r[...], preferred_element_type=jnp.float32)

    return pl.pallas_call(
        body,
        grid=(g,),
        in_specs=[
            pl.BlockSpec((nw_, rb, 128), lambda i: (0, i, 0)),
            pl.BlockSpec((blk, d), lambda i: (i, 0)),
            pl.BlockSpec((d, d), lambda i: (0, 0)),
        ],
        out_specs=[
            pl.BlockSpec((rb, 128), lambda i: (i, 0)),
            pl.BlockSpec((blk, d), lambda i: (i, 0)),
        ],
        out_shape=(jax.ShapeDtypeStruct((nrow, 128), jnp.float32),
                   jax.ShapeDtypeStruct((n_pad, d), jnp.float32)),
    )(degp, xp, w1)


def _tc_comb(p0, p1, xw, dis_col, w2=None):
    n_pad, d = xw.shape
    blk = 256
    g = n_pad // blk

    def body_mm(p0_r, p1_r, xw_r, dis_r, w2_r, o_r):
        dv = dis_r[...]
        h = jnp.maximum(p0_r[...] + p1_r[...] + dv * dv * xw_r[...], 0.0)
        o_r[...] = jnp.dot(h, w2_r[...], preferred_element_type=jnp.float32)

    def body_plain(p0_r, p1_r, xw_r, dis_r, o_r):
        dv = dis_r[...]
        o_r[...] = jnp.maximum(p0_r[...] + p1_r[...] + dv * dv * xw_r[...], 0.0)

    in_specs = [
        pl.BlockSpec((blk, d), lambda i: (i, 0)),
        pl.BlockSpec((blk, d), lambda i: (i, 0)),
        pl.BlockSpec((blk, d), lambda i: (i, 0)),
        pl.BlockSpec((blk, 1), lambda i: (i, 0)),
    ]
    args = [p0, p1, xw, dis_col]
    if w2 is not None:
        in_specs.append(pl.BlockSpec((d, d), lambda i: (0, 0)))
        args.append(w2)
        body = body_mm
    else:
        body = body_plain

    return pl.pallas_call(
        body,
        grid=(g,),
        in_specs=in_specs,
        out_specs=pl.BlockSpec((blk, d), lambda i: (i, 0)),
        out_shape=jax.ShapeDtypeStruct((n_pad, d), jnp.float32),
    )(*args)


def _tc_mlp(q, w0, b0, w1, b1, w2, b2, g0, be0, g1, be1):
    nq, d = q.shape
    h = w0.shape[0]
    c = w2.shape[0]
    blk = 512
    g = nq // blk
    cdims = (((1,), (1,)), ((), ()))

    def body(q_r, w0_r, b0_r, w1_r, b1_r, w2_r, b2_r,
             g0_r, be0_r, g1_r, be1_r, o_r):
        s = 1.0 / jnp.sqrt(jnp.float32(1.0 + 1e-5))
        x = lax.dot_general(q_r[...], w0_r[...], cdims,
                            preferred_element_type=jnp.float32) + b0_r[...]
        x = x * (g0_r[...] * s) + be0_r[...]
        x = jnp.maximum(x, 0.0)
        x = lax.dot_general(x, w1_r[...], cdims,
                            preferred_element_type=jnp.float32) + b1_r[...]
        x = x * (g1_r[...] * s) + be1_r[...]
        x = jnp.maximum(x, 0.0)
        x = lax.dot_general(x, w2_r[...], cdims,
                            preferred_element_type=jnp.float32) + b2_r[...]
        m = jnp.max(x, axis=-1, keepdims=True)
        ex = jnp.exp(x - m)
        o_r[...] = x - m - jnp.log(jnp.sum(ex, axis=-1, keepdims=True))

    full = lambda a: pl.BlockSpec(a.shape, lambda i: tuple(0 for _ in a.shape))
    args = [w0, b0.reshape(1, -1), w1, b1.reshape(1, -1), w2, b2.reshape(1, -1),
            g0.reshape(1, -1), be0.reshape(1, -1), g1.reshape(1, -1),
            be1.reshape(1, -1)]
    return pl.pallas_call(
        body,
        grid=(g,),
        in_specs=[pl.BlockSpec((blk, d), lambda i: (i, 0))] + [full(a) for a in args],
        out_specs=pl.BlockSpec((blk, c), lambda i: (i, 0)),
        out_shape=jax.ShapeDtypeStruct((nq, c), jnp.float32),
    )(q, *args)


def kernel(X, edge_index, edge_weight, edge1, edge2,
           iw1, w_ih1, w_hh1, b_ih1, b_hh1,
           iw2, w_ih2, w_hh2, b_ih2, b_hh2,
           lin0_w, lin0_b, lin1_w, lin1_b, lin2_w, lin2_b,
           bn0_g, bn0_b, bn1_g, bn1_b):
    n, d = X.shape
    e = edge_weight.shape[0]
    nq = edge1.shape[0]

    n_pad = _cdiv(n, NS * EB) * NS * EB
    e_pad = _cdiv(e, NW * EB * 2) * NW * EB * 2
    nq_pad = _cdiv(nq, NW * EB) * NW * EB

    row = jnp.pad(edge_index[0].astype(jnp.int32), (0, e_pad - e))
    col = jnp.pad(edge_index[1].astype(jnp.int32), (0, e_pad - e))
    w = jnp.pad(edge_weight.astype(jnp.float32), (0, e_pad - e))
    e1 = jnp.pad(edge1.astype(jnp.int32), (0, nq_pad - nq))
    e2 = jnp.pad(edge2.astype(jnp.int32), (0, nq_pad - nq))
    xp = jnp.pad(X, ((0, n_pad - n), (0, 0)))

    degp = _sc_deg(col, w, n_pad)
    w1, w2 = _tc_gru(iw1, w_ih1, w_hh1, b_ih1, b_hh1,
                     iw2, w_ih2, w_hh2, b_ih2, b_hh2)
    dis2d, xw1 = _tc_prep(degp, xp, w1)
    dis_flat = dis2d.reshape(-1)
    dis_col = dis_flat.reshape(-1, 1)

    norm = _sc_norm(row, col, w, dis_flat)
    row2 = row.reshape(-1, EB)
    col2 = col.reshape(-1, EB)
    nrm2 = norm.reshape(-1, EB)
    p = _sc_agg(xw1, row2, col2, nrm2)
    xw2 = _tc_comb(p[0], p[1], xw1, dis_col, w2)
    p2 = _sc_agg(xw2, row2, col2, nrm2)
    h2 = _tc_comb(p2[0], p2[1], xw2, dis_col)

    q = _sc_edge(h2, e1, e2)
    out = _tc_mlp(q, lin0_w, lin0_b, lin1_w, lin1_b, lin2_w, lin2_b,
                  bn0_g, bn0_b, bn1_g, bn1_b)
    return out[:nq]
